# baseline (speedup 1.0000x reference)
import functools

import jax
import jax.numpy as jnp
import numpy as np
from jax import lax
from jax.experimental import pallas as pl
from jax.experimental.pallas import tpu as pltpu
from jax.experimental.pallas import tpu_sc as plsc

_N = 32768
_K = 655
_L = 16
_NT = 16
_C = _N // _NT
_CV = _C // _L
_SIGN = np.uint32(0x80000000)


def _splat_i(x):
    return jnp.full((_L,), x, dtype=jnp.int32)


def _splat_f(x):
    return jnp.full((_L,), x, dtype=jnp.float32)


def _suffix(v):
    r = lax.rev(v, dimensions=(0,))
    return lax.rev(plsc.cumsum(r), dimensions=(0,))


def _key_of(v):
    u = lax.bitcast_convert_type(v, jnp.uint32)
    return jnp.where((u >> 31) == np.uint32(0), u | _SIGN, ~u)


def _val_of(k):
    u = jnp.where(k >= _SIGN, k ^ _SIGN, ~k)
    return lax.bitcast_convert_type(u, jnp.float32)


def _sc_body(x_hbm, out_hbm, xv, cand, rcs, ghl, outv, zvf, sem,
             gcs, gh1, gh2, gh3):
    sid = lax.axis_index("s")
    lanes = lax.iota(jnp.int32, _L)
    zeros_f = jnp.zeros((_L,), jnp.float32)
    ones_f = jnp.ones((_L,), jnp.float32)

    cp = pltpu.make_async_copy(x_hbm.at[pl.ds(sid * _C, _C)], xv, sem)
    cp.start()
    for g in range(3 * _L):
        zvf[g] = zeros_f
        rcs[g] = zeros_f

    @pl.when(sid == 0)
    def _():
        pltpu.sync_copy(zvf, gcs)
        pltpu.sync_copy(zvf.at[pl.ds(0, 2 * _L)], gh1)
        pltpu.sync_copy(zvf.at[pl.ds(0, 2 * _L)], gh2)
        pltpu.sync_copy(zvf.at[pl.ds(0, 2 * _L)], gh3)

    cp.wait()

    def r0(i, acc):
        v = xv[pl.ds(i * _L, _L)]
        key = _key_of(v)
        bhi = (key >> 28).astype(jnp.int32)
        blo = ((key >> 24) & np.uint32(0xF)).astype(jnp.int32)
        plsc.addupdate_scatter(rcs, [blo, bhi], ones_f)
        plsc.addupdate_scatter(rcs, [blo + _L, bhi], v)
        return acc + v * v

    sumsq_v = lax.fori_loop(0, _CV, r0, zeros_f, unroll=8)
    rcs[2 * _L] = sumsq_v
    plsc.subcore_barrier()
    pltpu.sync_copy(rcs.at[pl.ds(0, _L)], gcs.at[lanes], add=True)
    pltpu.sync_copy(rcs.at[pl.ds(_L, _L)], gcs.at[lanes + _L], add=True)
    pltpu.sync_copy(rcs.at[pl.ds(2 * _L, _L)], gcs.at[lanes + 2 * _L], add=True)
    plsc.subcore_barrier()
    pltpu.sync_copy(gcs, ghl)

    def scan(kr_v):
        ltot = zeros_f
        stot = zeros_f
        for g in range(_L):
            ltot = ltot + ghl[g]
            stot = stot + ghl[g + _L]
        sl = _suffix(ltot)
        l_sel = plsc.all_reduce_population_count(sl >= kr_v) - 1
        above1 = jnp.sum(jnp.where(lanes > l_sel, ltot, zeros_f))
        minor = plsc.load_gather(ghl, [lanes, l_sel])
        sminor = plsc.load_gather(ghl, [lanes + _L, l_sel])
        sm = _suffix(minor) + _splat_f(above1)
        c_sel = plsc.all_reduce_population_count(sm >= kr_v) - 1
        above2 = jnp.sum(jnp.where(lanes > c_sel, minor, zeros_f)) + above1
        sum_hi = (jnp.sum(jnp.where(lanes > l_sel, stot, zeros_f)) +
                  jnp.sum(jnp.where(lanes > c_sel, sminor, zeros_f)))
        return l_sel * _L + c_sel, _splat_f(above2), sum_hi

    kr_v = _splat_f(np.float32(_K))
    sel, above_v, sum_hi = scan(kr_v)
    kr_v = kr_v - above_v
    cnt_gt = jnp.max(above_v)
    sum_gt = sum_hi
    prefix_v = sel.astype(jnp.uint32) << 24
    sumsq = jnp.sum(ghl[2 * _L])

    for g in range(2 * _L):
        rcs[g] = zeros_f

    def comp(i, off_v):
        v = xv[pl.ds(i * _L, _L)]
        k = _key_of(v)
        m = (k >> 24) == (prefix_v >> 24)
        pc = plsc.cumsum(m.astype(jnp.int32))
        plsc.store_scatter(cand, [off_v + pc - 1],
                           lax.bitcast_convert_type(k, jnp.int32), mask=m)
        bhi = ((k >> 20) & np.uint32(0xF)).astype(jnp.int32)
        blo = ((k >> 16) & np.uint32(0xF)).astype(jnp.int32)
        plsc.addupdate_scatter(rcs, [blo, bhi], ones_f, mask=m)
        plsc.addupdate_scatter(rcs, [blo + _L, bhi], v, mask=m)
        return off_v + plsc.all_reduce_population_count(m)

    nc_v = lax.fori_loop(0, _CV, comp, _splat_i(0), unroll=8)
    nvec = (jnp.max(nc_v) + _L - 1) // _L

    for rnd, (gh, shift) in enumerate(((gh1, 16), (gh2, 8), (gh3, 0))):
        if rnd > 0:
            for g in range(2 * _L):
                rcs[g] = zeros_f

            def rr(i, c, shift=shift, prefix_v=prefix_v):
                k = lax.bitcast_convert_type(cand[pl.ds(i * _L, _L)],
                                             jnp.uint32)
                valid = (i * _L + lanes) < nc_v
                m = (((k ^ prefix_v) >> (shift + 8)) == np.uint32(0)) & valid
                bhi = ((k >> (shift + 4)) & np.uint32(0xF)).astype(jnp.int32)
                blo = ((k >> shift) & np.uint32(0xF)).astype(jnp.int32)
                plsc.addupdate_scatter(rcs, [blo, bhi], ones_f, mask=m)
                plsc.addupdate_scatter(rcs, [blo + _L, bhi], _val_of(k),
                                       mask=m)
                return c

            lax.fori_loop(0, nvec, rr, 0)
        pltpu.sync_copy(rcs.at[pl.ds(0, _L)], gh.at[lanes], add=True)
        pltpu.sync_copy(rcs.at[pl.ds(_L, _L)], gh.at[lanes + _L], add=True)
        plsc.subcore_barrier()
        pltpu.sync_copy(gh, ghl.at[pl.ds(0, 2 * _L)])
        sel, above_v, sum_hi = scan(kr_v)
        kr_v = kr_v - above_v
        cnt_gt = cnt_gt + jnp.max(above_v)
        sum_gt = sum_gt + sum_hi
        prefix_v = prefix_v | (sel.astype(jnp.uint32) << shift)

    val_t = _val_of(prefix_v)
    top = _splat_f(sum_gt) + (_splat_f(np.float32(_K)) - _splat_f(cnt_gt)) * val_t

    svec = _splat_f(sumsq)
    i0 = np.uint32(0x5F3759DF) - (lax.bitcast_convert_type(svec, jnp.uint32) >> 1)
    y = lax.bitcast_convert_type(i0, jnp.float32)
    for _ in range(3):
        y = y * (1.5 - 0.5 * svec * y * y)
    norm = jnp.maximum(svec * y, _splat_f(np.float32(1e-12)))
    outv[...] = jnp.where(svec > 0, top / norm, zeros_f)

    @pl.when(sid == 0)
    def _():
        pltpu.sync_copy(outv, out_hbm)


_topk_sum_sc = functools.partial(
    pl.kernel,
    out_type=jax.ShapeDtypeStruct((_L,), jnp.float32),
    mesh=plsc.VectorSubcoreMesh(
        core_axis_name="c", subcore_axis_name="s",
        num_cores=1, num_subcores=16),
    compiler_params=pltpu.CompilerParams(
        needs_layout_passes=False, use_tc_tiling_on_sc=False),
    scratch_types=[
        pltpu.VMEM((_C,), jnp.float32),
        pltpu.VMEM((_C,), jnp.int32),
        pltpu.VMEM((3 * _L, _L), jnp.float32),
        pltpu.VMEM((3 * _L, _L), jnp.float32),
        pltpu.VMEM((_L,), jnp.float32),
        pltpu.VMEM((3 * _L, _L), jnp.float32),
        pltpu.SemaphoreType.DMA,
        pltpu.VMEM_SHARED((3 * _L, _L), jnp.float32),
        pltpu.VMEM_SHARED((2 * _L, _L), jnp.float32),
        pltpu.VMEM_SHARED((2 * _L, _L), jnp.float32),
        pltpu.VMEM_SHARED((2 * _L, _L), jnp.float32),
    ],
)(_sc_body)


def kernel(glb_feature, aux):
    x = jnp.reshape(glb_feature, (_N,))
    return _topk_sum_sc(x)[0]


# speedup vs baseline: 1.0212x; 1.0212x over previous
import functools

import jax
import jax.numpy as jnp
import numpy as np
from jax import lax
from jax.experimental import pallas as pl
from jax.experimental.pallas import tpu as pltpu
from jax.experimental.pallas import tpu_sc as plsc

_N = 32768
_K = 655
_L = 16
_NT = 16
_C = _N // _NT
_CV = _C // _L
_SIGN = np.uint32(0x80000000)


def _splat_i(x):
    return jnp.full((_L,), x, dtype=jnp.int32)


def _splat_f(x):
    return jnp.full((_L,), x, dtype=jnp.float32)


def _suffix(v):
    r = lax.rev(v, dimensions=(0,))
    return lax.rev(plsc.cumsum(r), dimensions=(0,))


def _key_of(v):
    u = lax.bitcast_convert_type(v, jnp.uint32)
    return jnp.where((u >> 31) == np.uint32(0), u | _SIGN, ~u)


def _val_of(k):
    u = jnp.where(k >= _SIGN, k ^ _SIGN, ~k)
    return lax.bitcast_convert_type(u, jnp.float32)


def _sc_body(x_hbm, out_hbm, xv, cand, rcs, ghl, outv, zvf, sem,
             gcs, gh1, gh2, gh3):
    sid = lax.axis_index("s")
    lanes = lax.iota(jnp.int32, _L)
    zeros_f = jnp.zeros((_L,), jnp.float32)
    ones_f = jnp.ones((_L,), jnp.float32)

    cp = pltpu.make_async_copy(x_hbm.at[pl.ds(sid * _C, _C)], xv, sem)
    cp.start()
    for g in range(3 * _L):
        zvf[g] = zeros_f
        rcs[g] = zeros_f

    @pl.when(sid == 0)
    def _():
        pltpu.sync_copy(zvf, gcs)
        pltpu.sync_copy(zvf.at[pl.ds(0, 2 * _L)], gh1)
        pltpu.sync_copy(zvf.at[pl.ds(0, 2 * _L)], gh2)
        pltpu.sync_copy(zvf.at[pl.ds(0, 2 * _L)], gh3)

    cp.wait()

    def r0(i, acc):
        v = xv[pl.ds(i * _L, _L)]
        key = _key_of(v)
        bhi = (key >> 28).astype(jnp.int32)
        blo = ((key >> 24) & np.uint32(0xF)).astype(jnp.int32)
        plsc.addupdate_scatter(rcs, [blo, bhi], ones_f)
        plsc.addupdate_scatter(rcs, [blo + _L, bhi], v)
        return acc + v * v

    sumsq_v = lax.fori_loop(0, _CV, r0, zeros_f, unroll=2)
    rcs[2 * _L] = sumsq_v
    plsc.subcore_barrier()
    pltpu.sync_copy(rcs.at[pl.ds(0, _L)], gcs.at[lanes], add=True)
    pltpu.sync_copy(rcs.at[pl.ds(_L, _L)], gcs.at[lanes + _L], add=True)
    pltpu.sync_copy(rcs.at[pl.ds(2 * _L, _L)], gcs.at[lanes + 2 * _L], add=True)
    plsc.subcore_barrier()
    pltpu.sync_copy(gcs, ghl)

    def scan(kr_v):
        ltot = zeros_f
        stot = zeros_f
        for g in range(_L):
            ltot = ltot + ghl[g]
            stot = stot + ghl[g + _L]
        sl = _suffix(ltot)
        l_sel = plsc.all_reduce_population_count(sl >= kr_v) - 1
        above1 = jnp.sum(jnp.where(lanes > l_sel, ltot, zeros_f))
        minor = plsc.load_gather(ghl, [lanes, l_sel])
        sminor = plsc.load_gather(ghl, [lanes + _L, l_sel])
        sm = _suffix(minor) + _splat_f(above1)
        c_sel = plsc.all_reduce_population_count(sm >= kr_v) - 1
        above2 = jnp.sum(jnp.where(lanes > c_sel, minor, zeros_f)) + above1
        sum_hi = (jnp.sum(jnp.where(lanes > l_sel, stot, zeros_f)) +
                  jnp.sum(jnp.where(lanes > c_sel, sminor, zeros_f)))
        return l_sel * _L + c_sel, _splat_f(above2), sum_hi

    kr_v = _splat_f(np.float32(_K))
    sel, above_v, sum_hi = scan(kr_v)
    kr_v = kr_v - above_v
    cnt_gt = jnp.max(above_v)
    sum_gt = sum_hi
    prefix_v = sel.astype(jnp.uint32) << 24
    sumsq = jnp.sum(ghl[2 * _L])

    for g in range(2 * _L):
        rcs[g] = zeros_f

    def comp(i, off_v):
        v = xv[pl.ds(i * _L, _L)]
        k = _key_of(v)
        m = (k >> 24) == (prefix_v >> 24)
        pc = plsc.cumsum(m.astype(jnp.int32))
        plsc.store_scatter(cand, [off_v + pc - 1],
                           lax.bitcast_convert_type(k, jnp.int32), mask=m)
        bhi = ((k >> 20) & np.uint32(0xF)).astype(jnp.int32)
        blo = ((k >> 16) & np.uint32(0xF)).astype(jnp.int32)
        plsc.addupdate_scatter(rcs, [blo, bhi], ones_f, mask=m)
        plsc.addupdate_scatter(rcs, [blo + _L, bhi], v, mask=m)
        return off_v + plsc.all_reduce_population_count(m)

    nc_v = lax.fori_loop(0, _CV, comp, _splat_i(0), unroll=2)
    nvec = (jnp.max(nc_v) + _L - 1) // _L

    for rnd, (gh, shift) in enumerate(((gh1, 16), (gh2, 8), (gh3, 0))):
        if rnd > 0:
            for g in range(2 * _L):
                rcs[g] = zeros_f

            def rr(i, c, shift=shift, prefix_v=prefix_v):
                k = lax.bitcast_convert_type(cand[pl.ds(i * _L, _L)],
                                             jnp.uint32)
                valid = (i * _L + lanes) < nc_v
                m = (((k ^ prefix_v) >> (shift + 8)) == np.uint32(0)) & valid
                bhi = ((k >> (shift + 4)) & np.uint32(0xF)).astype(jnp.int32)
                blo = ((k >> shift) & np.uint32(0xF)).astype(jnp.int32)
                plsc.addupdate_scatter(rcs, [blo, bhi], ones_f, mask=m)
                plsc.addupdate_scatter(rcs, [blo + _L, bhi], _val_of(k),
                                       mask=m)
                return c

            lax.fori_loop(0, nvec, rr, 0)
        pltpu.sync_copy(rcs.at[pl.ds(0, _L)], gh.at[lanes], add=True)
        pltpu.sync_copy(rcs.at[pl.ds(_L, _L)], gh.at[lanes + _L], add=True)
        plsc.subcore_barrier()
        pltpu.sync_copy(gh, ghl.at[pl.ds(0, 2 * _L)])
        sel, above_v, sum_hi = scan(kr_v)
        kr_v = kr_v - above_v
        cnt_gt = cnt_gt + jnp.max(above_v)
        sum_gt = sum_gt + sum_hi
        prefix_v = prefix_v | (sel.astype(jnp.uint32) << shift)

    val_t = _val_of(prefix_v)
    top = _splat_f(sum_gt) + (_splat_f(np.float32(_K)) - _splat_f(cnt_gt)) * val_t

    svec = _splat_f(sumsq)
    i0 = np.uint32(0x5F3759DF) - (lax.bitcast_convert_type(svec, jnp.uint32) >> 1)
    y = lax.bitcast_convert_type(i0, jnp.float32)
    for _ in range(3):
        y = y * (1.5 - 0.5 * svec * y * y)
    norm = jnp.maximum(svec * y, _splat_f(np.float32(1e-12)))
    outv[...] = jnp.where(svec > 0, top / norm, zeros_f)

    @pl.when(sid == 0)
    def _():
        pltpu.sync_copy(outv, out_hbm)


_topk_sum_sc = functools.partial(
    pl.kernel,
    out_type=jax.ShapeDtypeStruct((_L,), jnp.float32),
    mesh=plsc.VectorSubcoreMesh(
        core_axis_name="c", subcore_axis_name="s",
        num_cores=1, num_subcores=16),
    compiler_params=pltpu.CompilerParams(
        needs_layout_passes=False, use_tc_tiling_on_sc=False),
    scratch_types=[
        pltpu.VMEM((_C,), jnp.float32),
        pltpu.VMEM((_C,), jnp.int32),
        pltpu.VMEM((3 * _L, _L), jnp.float32),
        pltpu.VMEM((3 * _L, _L), jnp.float32),
        pltpu.VMEM((_L,), jnp.float32),
        pltpu.VMEM((3 * _L, _L), jnp.float32),
        pltpu.SemaphoreType.DMA,
        pltpu.VMEM_SHARED((3 * _L, _L), jnp.float32),
        pltpu.VMEM_SHARED((2 * _L, _L), jnp.float32),
        pltpu.VMEM_SHARED((2 * _L, _L), jnp.float32),
        pltpu.VMEM_SHARED((2 * _L, _L), jnp.float32),
    ],
)(_sc_body)


def kernel(glb_feature, aux):
    x = jnp.reshape(glb_feature, (_N,))
    return _topk_sum_sc(x)[0]


# speedup vs baseline: 1.0213x; 1.0001x over previous
import functools

import jax
import jax.numpy as jnp
import numpy as np
from jax import lax
from jax.experimental import pallas as pl
from jax.experimental.pallas import tpu as pltpu
from jax.experimental.pallas import tpu_sc as plsc

_N = 32768
_K = 655
_L = 16
_NT = 16
_C = _N // _NT
_CV = _C // _L
_SIGN = np.uint32(0x80000000)


def _splat_i(x):
    return jnp.full((_L,), x, dtype=jnp.int32)


def _splat_f(x):
    return jnp.full((_L,), x, dtype=jnp.float32)


def _suffix(v):
    r = lax.rev(v, dimensions=(0,))
    return lax.rev(plsc.cumsum(r), dimensions=(0,))


def _key_of(v):
    u = lax.bitcast_convert_type(v, jnp.uint32)
    return jnp.where((u >> 31) == np.uint32(0), u | _SIGN, ~u)


def _val_of(k):
    u = jnp.where(k >= _SIGN, k ^ _SIGN, ~k)
    return lax.bitcast_convert_type(u, jnp.float32)


def _sc_body(x_hbm, out_hbm, xv, cand, rcs, ghl, outv, zvf, sem,
             gcs, gh1, gh2, gh3):
    sid = lax.axis_index("s")
    lanes = lax.iota(jnp.int32, _L)
    zeros_f = jnp.zeros((_L,), jnp.float32)
    ones_f = jnp.ones((_L,), jnp.float32)

    cp = pltpu.make_async_copy(x_hbm.at[pl.ds(sid * _C, _C)], xv, sem)
    cp.start()
    for g in range(3 * _L):
        zvf[g] = zeros_f
        rcs[g] = zeros_f

    @pl.when(sid == 0)
    def _():
        pltpu.sync_copy(zvf, gcs)
        pltpu.sync_copy(zvf.at[pl.ds(0, 2 * _L)], gh1)
        pltpu.sync_copy(zvf.at[pl.ds(0, 2 * _L)], gh2)
        pltpu.sync_copy(zvf.at[pl.ds(0, 2 * _L)], gh3)

    cp.wait()

    def r0(i, acc):
        v = xv[pl.ds(i * _L, _L)]
        key = _key_of(v)
        bhi = (key >> 28).astype(jnp.int32)
        blo = ((key >> 24) & np.uint32(0xF)).astype(jnp.int32)
        plsc.addupdate_scatter(rcs, [blo, bhi], ones_f)
        plsc.addupdate_scatter(rcs, [blo + _L, bhi], v)
        return acc + v * v

    with jax.named_scope("ph_r0"):
        sumsq_v = lax.fori_loop(0, _CV, r0, zeros_f, unroll=2)
    rcs[2 * _L] = sumsq_v
    with jax.named_scope("ph_merge0"):
        plsc.subcore_barrier()
        pltpu.sync_copy(rcs.at[pl.ds(0, _L)], gcs.at[lanes], add=True)
        pltpu.sync_copy(rcs.at[pl.ds(_L, _L)], gcs.at[lanes + _L], add=True)
        pltpu.sync_copy(rcs.at[pl.ds(2 * _L, _L)], gcs.at[lanes + 2 * _L], add=True)
        plsc.subcore_barrier()
        pltpu.sync_copy(gcs, ghl)

    def scan(kr_v):
        ltot = zeros_f
        stot = zeros_f
        for g in range(_L):
            ltot = ltot + ghl[g]
            stot = stot + ghl[g + _L]
        sl = _suffix(ltot)
        l_sel = plsc.all_reduce_population_count(sl >= kr_v) - 1
        above1 = jnp.sum(jnp.where(lanes > l_sel, ltot, zeros_f))
        minor = plsc.load_gather(ghl, [lanes, l_sel])
        sminor = plsc.load_gather(ghl, [lanes + _L, l_sel])
        sm = _suffix(minor) + _splat_f(above1)
        c_sel = plsc.all_reduce_population_count(sm >= kr_v) - 1
        above2 = jnp.sum(jnp.where(lanes > c_sel, minor, zeros_f)) + above1
        sum_hi = (jnp.sum(jnp.where(lanes > l_sel, stot, zeros_f)) +
                  jnp.sum(jnp.where(lanes > c_sel, sminor, zeros_f)))
        return l_sel * _L + c_sel, _splat_f(above2), sum_hi

    kr_v = _splat_f(np.float32(_K))
    with jax.named_scope("ph_scan0"):
        sel, above_v, sum_hi = scan(kr_v)
    kr_v = kr_v - above_v
    cnt_gt = jnp.max(above_v)
    sum_gt = sum_hi
    prefix_v = sel.astype(jnp.uint32) << 24
    sumsq = jnp.sum(ghl[2 * _L])

    for g in range(2 * _L):
        rcs[g] = zeros_f

    def comp(i, off_v):
        v = xv[pl.ds(i * _L, _L)]
        k = _key_of(v)
        m = (k >> 24) == (prefix_v >> 24)
        pc = plsc.cumsum(m.astype(jnp.int32))
        plsc.store_scatter(cand, [off_v + pc - 1],
                           lax.bitcast_convert_type(k, jnp.int32), mask=m)
        bhi = ((k >> 20) & np.uint32(0xF)).astype(jnp.int32)
        blo = ((k >> 16) & np.uint32(0xF)).astype(jnp.int32)
        plsc.addupdate_scatter(rcs, [blo, bhi], ones_f, mask=m)
        plsc.addupdate_scatter(rcs, [blo + _L, bhi], v, mask=m)
        return off_v + plsc.all_reduce_population_count(m)

    with jax.named_scope("ph_comp"):
        nc_v = lax.fori_loop(0, _CV, comp, _splat_i(0), unroll=2)
    nvec = (jnp.max(nc_v) + _L - 1) // _L

    for rnd, (gh, shift) in enumerate(((gh1, 16), (gh2, 8), (gh3, 0))):
        if rnd > 0:
            for g in range(2 * _L):
                rcs[g] = zeros_f

            def rr(i, c, shift=shift, prefix_v=prefix_v):
                k = lax.bitcast_convert_type(cand[pl.ds(i * _L, _L)],
                                             jnp.uint32)
                valid = (i * _L + lanes) < nc_v
                m = (((k ^ prefix_v) >> (shift + 8)) == np.uint32(0)) & valid
                bhi = ((k >> (shift + 4)) & np.uint32(0xF)).astype(jnp.int32)
                blo = ((k >> shift) & np.uint32(0xF)).astype(jnp.int32)
                plsc.addupdate_scatter(rcs, [blo, bhi], ones_f, mask=m)
                plsc.addupdate_scatter(rcs, [blo + _L, bhi], _val_of(k),
                                       mask=m)
                return c

            with jax.named_scope("ph_rr"):
                lax.fori_loop(0, nvec, rr, 0)
        with jax.named_scope("ph_rmerge"):
            pltpu.sync_copy(rcs.at[pl.ds(0, _L)], gh.at[lanes], add=True)
            pltpu.sync_copy(rcs.at[pl.ds(_L, _L)], gh.at[lanes + _L], add=True)
            plsc.subcore_barrier()
            pltpu.sync_copy(gh, ghl.at[pl.ds(0, 2 * _L)])
        with jax.named_scope("ph_rscan"):
            sel, above_v, sum_hi = scan(kr_v)
        kr_v = kr_v - above_v
        cnt_gt = cnt_gt + jnp.max(above_v)
        sum_gt = sum_gt + sum_hi
        prefix_v = prefix_v | (sel.astype(jnp.uint32) << shift)

    val_t = _val_of(prefix_v)
    top = _splat_f(sum_gt) + (_splat_f(np.float32(_K)) - _splat_f(cnt_gt)) * val_t

    svec = _splat_f(sumsq)
    i0 = np.uint32(0x5F3759DF) - (lax.bitcast_convert_type(svec, jnp.uint32) >> 1)
    y = lax.bitcast_convert_type(i0, jnp.float32)
    for _ in range(3):
        y = y * (1.5 - 0.5 * svec * y * y)
    norm = jnp.maximum(svec * y, _splat_f(np.float32(1e-12)))
    outv[...] = jnp.where(svec > 0, top / norm, zeros_f)

    @pl.when(sid == 0)
    def _():
        pltpu.sync_copy(outv, out_hbm)


_topk_sum_sc = functools.partial(
    pl.kernel,
    out_type=jax.ShapeDtypeStruct((_L,), jnp.float32),
    mesh=plsc.VectorSubcoreMesh(
        core_axis_name="c", subcore_axis_name="s",
        num_cores=1, num_subcores=16),
    compiler_params=pltpu.CompilerParams(
        needs_layout_passes=False, use_tc_tiling_on_sc=False),
    scratch_types=[
        pltpu.VMEM((_C,), jnp.float32),
        pltpu.VMEM((_C,), jnp.int32),
        pltpu.VMEM((3 * _L, _L), jnp.float32),
        pltpu.VMEM((3 * _L, _L), jnp.float32),
        pltpu.VMEM((_L,), jnp.float32),
        pltpu.VMEM((3 * _L, _L), jnp.float32),
        pltpu.SemaphoreType.DMA,
        pltpu.VMEM_SHARED((3 * _L, _L), jnp.float32),
        pltpu.VMEM_SHARED((2 * _L, _L), jnp.float32),
        pltpu.VMEM_SHARED((2 * _L, _L), jnp.float32),
        pltpu.VMEM_SHARED((2 * _L, _L), jnp.float32),
    ],
)(_sc_body)


def kernel(glb_feature, aux):
    x = jnp.reshape(glb_feature, (_N,))
    return _topk_sum_sc(x)[0]


# speedup vs baseline: 1.0511x; 1.0292x over previous
import functools

import jax
import jax.numpy as jnp
import numpy as np
from jax import lax
from jax.experimental import pallas as pl
from jax.experimental.pallas import tpu as pltpu
from jax.experimental.pallas import tpu_sc as plsc

_N = 32768
_K = 655
_L = 16
_NT = 16
_C = _N // _NT
_CV = _C // _L
_SIGN = np.uint32(0x80000000)


def _splat_i(x):
    return jnp.full((_L,), x, dtype=jnp.int32)


def _splat_f(x):
    return jnp.full((_L,), x, dtype=jnp.float32)


def _suffix(v):
    r = lax.rev(v, dimensions=(0,))
    return lax.rev(plsc.cumsum(r), dimensions=(0,))


def _key_of(v):
    u = lax.bitcast_convert_type(v, jnp.uint32)
    return jnp.where((u >> 31) == np.uint32(0), u | _SIGN, ~u)


def _val_of(k):
    u = jnp.where(k >= _SIGN, k ^ _SIGN, ~k)
    return lax.bitcast_convert_type(u, jnp.float32)


def _sc_body(x_hbm, out_hbm, xv, cand, rcs, ghl, outv, zvf, sem,
             gcs, gh1, gh2, gh3):
    sid = lax.axis_index("s")
    lanes = lax.iota(jnp.int32, _L)
    zeros_f = jnp.zeros((_L,), jnp.float32)
    ones_f = jnp.ones((_L,), jnp.float32)

    cp = pltpu.make_async_copy(x_hbm.at[pl.ds(sid * _C, _C)], xv, sem)
    cp.start()
    for g in range(3 * _L):
        zvf[g] = zeros_f
        rcs[g] = zeros_f

    @pl.when(sid == 0)
    def _():
        pltpu.sync_copy(zvf, gcs)
        pltpu.sync_copy(zvf.at[pl.ds(0, 2 * _L)], gh1)
        pltpu.sync_copy(zvf.at[pl.ds(0, 2 * _L)], gh2)
        pltpu.sync_copy(zvf.at[pl.ds(0, 2 * _L)], gh3)

    cp.wait()

    def r0(i, acc):
        v = xv[pl.ds(i * _L, _L)]
        key = _key_of(v)
        bhi = (key >> 28).astype(jnp.int32)
        blo = ((key >> 24) & np.uint32(0xF)).astype(jnp.int32)
        plsc.addupdate_scatter(rcs, [blo, bhi], ones_f)
        return acc + v * v

    with jax.named_scope("ph_r0"):
        sumsq_v = lax.fori_loop(0, _CV, r0, zeros_f, unroll=4)
    rcs[2 * _L] = sumsq_v
    with jax.named_scope("ph_merge0"):
        plsc.subcore_barrier()
        pltpu.sync_copy(rcs.at[pl.ds(0, _L)], gcs.at[lanes], add=True)
        pltpu.sync_copy(rcs.at[pl.ds(2 * _L, _L)], gcs.at[lanes + 2 * _L], add=True)
        plsc.subcore_barrier()
        pltpu.sync_copy(gcs, ghl)

    def scan(kr_v):
        ltot = zeros_f
        for g in range(_L):
            ltot = ltot + ghl[g]
        sl = _suffix(ltot)
        l_sel = plsc.all_reduce_population_count(sl >= kr_v) - 1
        above1 = jnp.sum(jnp.where(lanes > l_sel, ltot, zeros_f))
        minor = plsc.load_gather(ghl, [lanes, l_sel])
        sm = _suffix(minor) + _splat_f(above1)
        c_sel = plsc.all_reduce_population_count(sm >= kr_v) - 1
        above2 = jnp.sum(jnp.where(lanes > c_sel, minor, zeros_f)) + above1
        return l_sel * _L + c_sel, _splat_f(above2)

    kr_v = _splat_f(np.float32(_K))
    with jax.named_scope("ph_scan0"):
        sel, above_v = scan(kr_v)
    kr_v = kr_v - above_v
    cnt_gt = jnp.max(above_v)
    prefix_v = sel.astype(jnp.uint32) << 24
    sumsq = jnp.sum(ghl[2 * _L])

    for g in range(_L):
        rcs[g] = zeros_f

    def comp(i, off_v):
        v = xv[pl.ds(i * _L, _L)]
        k = _key_of(v)
        m = (k >> 24) == (prefix_v >> 24)
        pc = plsc.cumsum(m.astype(jnp.int32))
        plsc.store_scatter(cand, [off_v + pc - 1],
                           lax.bitcast_convert_type(k, jnp.int32), mask=m)
        bhi = ((k >> 20) & np.uint32(0xF)).astype(jnp.int32)
        blo = ((k >> 16) & np.uint32(0xF)).astype(jnp.int32)
        plsc.addupdate_scatter(rcs, [blo, bhi], ones_f, mask=m)
        return off_v + plsc.all_reduce_population_count(m)

    with jax.named_scope("ph_comp"):
        nc_v = lax.fori_loop(0, _CV, comp, _splat_i(0), unroll=4)
    nvec = (jnp.max(nc_v) + _L - 1) // _L

    for rnd, (gh, shift) in enumerate(((gh1, 16), (gh2, 8), (gh3, 0))):
        if rnd > 0:
            for g in range(_L):
                rcs[g] = zeros_f

            def rr(i, c, shift=shift, prefix_v=prefix_v):
                k = lax.bitcast_convert_type(cand[pl.ds(i * _L, _L)],
                                             jnp.uint32)
                valid = (i * _L + lanes) < nc_v
                m = (((k ^ prefix_v) >> (shift + 8)) == np.uint32(0)) & valid
                bhi = ((k >> (shift + 4)) & np.uint32(0xF)).astype(jnp.int32)
                blo = ((k >> shift) & np.uint32(0xF)).astype(jnp.int32)
                plsc.addupdate_scatter(rcs, [blo, bhi], ones_f, mask=m)
                return c

            with jax.named_scope("ph_rr"):
                lax.fori_loop(0, nvec, rr, 0)
        with jax.named_scope("ph_rmerge"):
            pltpu.sync_copy(rcs.at[pl.ds(0, _L)], gh.at[lanes], add=True)
            plsc.subcore_barrier()
            pltpu.sync_copy(gh.at[pl.ds(0, _L)], ghl.at[pl.ds(0, _L)])
        with jax.named_scope("ph_rscan"):
            sel, above_v = scan(kr_v)
        kr_v = kr_v - above_v
        cnt_gt = cnt_gt + jnp.max(above_v)
        prefix_v = prefix_v | (sel.astype(jnp.uint32) << shift)

    def fsum(i, acc, prefix_v=prefix_v):
        v = xv[pl.ds(i * _L, _L)]
        return acc + jnp.where(_key_of(v) > prefix_v, v, zeros_f)

    with jax.named_scope("ph_fsum"):
        sum_part = lax.fori_loop(0, _CV, fsum, zeros_f, unroll=4)
    rcs[0] = sum_part
    for g in range(1, _L):
        rcs[g] = zeros_f
    with jax.named_scope("ph_fmerge"):
        pltpu.sync_copy(rcs.at[pl.ds(0, _L)], gcs.at[lanes + _L], add=True)
        plsc.subcore_barrier()
        pltpu.sync_copy(gcs.at[pl.ds(_L, _L)], ghl.at[pl.ds(0, _L)])
    sum_gt_v = zeros_f
    for g in range(_L):
        sum_gt_v = sum_gt_v + ghl[g]
    sum_gt = jnp.sum(sum_gt_v)

    val_t = _val_of(prefix_v)
    top = _splat_f(sum_gt) + (_splat_f(np.float32(_K)) - _splat_f(cnt_gt)) * val_t

    svec = _splat_f(sumsq)
    i0 = np.uint32(0x5F3759DF) - (lax.bitcast_convert_type(svec, jnp.uint32) >> 1)
    y = lax.bitcast_convert_type(i0, jnp.float32)
    for _ in range(3):
        y = y * (1.5 - 0.5 * svec * y * y)
    norm = jnp.maximum(svec * y, _splat_f(np.float32(1e-12)))
    outv[...] = jnp.where(svec > 0, top / norm, zeros_f)

    @pl.when(sid == 0)
    def _():
        pltpu.sync_copy(outv, out_hbm)


_topk_sum_sc = functools.partial(
    pl.kernel,
    out_type=jax.ShapeDtypeStruct((_L,), jnp.float32),
    mesh=plsc.VectorSubcoreMesh(
        core_axis_name="c", subcore_axis_name="s",
        num_cores=1, num_subcores=16),
    compiler_params=pltpu.CompilerParams(
        needs_layout_passes=False, use_tc_tiling_on_sc=False),
    scratch_types=[
        pltpu.VMEM((_C,), jnp.float32),
        pltpu.VMEM((_C,), jnp.int32),
        pltpu.VMEM((3 * _L, _L), jnp.float32),
        pltpu.VMEM((3 * _L, _L), jnp.float32),
        pltpu.VMEM((_L,), jnp.float32),
        pltpu.VMEM((3 * _L, _L), jnp.float32),
        pltpu.SemaphoreType.DMA,
        pltpu.VMEM_SHARED((3 * _L, _L), jnp.float32),
        pltpu.VMEM_SHARED((2 * _L, _L), jnp.float32),
        pltpu.VMEM_SHARED((2 * _L, _L), jnp.float32),
        pltpu.VMEM_SHARED((2 * _L, _L), jnp.float32),
    ],
)(_sc_body)


def kernel(glb_feature, aux):
    x = jnp.reshape(glb_feature, (_N,))
    return _topk_sum_sc(x)[0]
